# baseline (speedup 1.0000x reference)
import functools

import jax
import jax.numpy as jnp
from jax import lax
from jax.experimental import pallas as pl
from jax.experimental.pallas import tpu as pltpu
from jax.experimental.pallas import tpu_sc as plsc

_ROWS = 4
_NBUF = 2


@functools.cache
def _make_lookup(batch, seq, V, D):
    info = plsc.get_sparse_core_info()
    nc, ns = info.num_cores, info.num_subcores
    nw = nc * ns
    rows_per_w = batch // nw
    n_super = rows_per_w // (_ROWS * _NBUF)
    n_half = n_super // 2
    assert rows_per_w == n_half * 2 * _ROWS * _NBUF
    mesh = plsc.VectorSubcoreMesh(core_axis_name="c", subcore_axis_name="s")

    @functools.partial(
        pl.kernel,
        out_type=jax.ShapeDtypeStruct((batch, seq, 128), jnp.float32),
        mesh=mesh,
        scratch_types=[
            pltpu.VMEM((2, _NBUF, _ROWS, seq), jnp.int32),
            pltpu.VMEM((_NBUF, _ROWS, seq, D), jnp.float32),
            pltpu.SemaphoreType.DMA((2, _NBUF)),
            pltpu.SemaphoreType.DMA((_NBUF,)),
            pltpu.SemaphoreType.DMA((_NBUF,)),
        ],
        compiler_params=pltpu.CompilerParams(use_tc_tiling_on_sc=False),
    )
    def lookup(ids_hbm, table_hbm, out_hbm, idx_v, rows_v, isem, gsem, wsem):
        wid = lax.axis_index("s") * nc + lax.axis_index("c")
        base = wid * rows_per_w

        def chunk_row(i, b):
            return base + (i * _NBUF + b) * _ROWS

        def prefetch_ids(i, p):
            for b in range(_NBUF):
                pltpu.async_copy(
                    ids_hbm.at[pl.ds(chunk_row(i, b), _ROWS)],
                    idx_v.at[p, b],
                    isem.at[p, b],
                )

        prefetch_ids(0, 0)

        def super_pair(j, carry):
            for p in range(2):
                i = 2 * j + p
                for b in range(_NBUF):
                    row = chunk_row(i, b)

                    @pl.when(i > 0)
                    def _drain(b=b, row=row):
                        pltpu.make_async_copy(
                            rows_v.at[b],
                            out_hbm.at[pl.ds(row, _ROWS), :, pl.ds(0, D)],
                            wsem.at[b],
                        ).wait()

                    pltpu.make_async_copy(
                        ids_hbm.at[pl.ds(row, _ROWS)], idx_v.at[p, b], isem.at[p, b]
                    ).wait()
                    for r in range(_ROWS):
                        pltpu.async_copy(
                            table_hbm.at[idx_v.at[p, b, r]], rows_v.at[b, r], gsem.at[b]
                        )
                if p == 0:
                    prefetch_ids(i + 1, 1)
                else:

                    @pl.when(j < n_half - 1)
                    def _next_ids(i=i):
                        prefetch_ids(i + 1, 0)

                for b in range(_NBUF):
                    row = chunk_row(i, b)
                    pltpu.make_async_copy(
                        table_hbm.at[idx_v.at[p, b, 0]], rows_v.at[b], gsem.at[b]
                    ).wait()
                    pltpu.async_copy(
                        rows_v.at[b],
                        out_hbm.at[pl.ds(row, _ROWS), :, pl.ds(0, D)],
                        wsem.at[b],
                    )
            return carry

        lax.fori_loop(0, n_half, super_pair, 0)
        for b in range(_NBUF):
            pltpu.make_async_copy(
                rows_v.at[b],
                out_hbm.at[pl.ds(base, _ROWS), :, pl.ds(0, D)],
                wsem.at[b],
            ).wait()

    return lookup


def kernel(token_ids, weights):
    batch, seq = token_ids.shape
    vocab, d = weights.shape
    ids = token_ids.astype(jnp.int32)
    out_pad = _make_lookup(batch, seq, vocab, d)(ids, weights)
    return out_pad[:, :, :d]


# speedup vs baseline: 1.0035x; 1.0035x over previous
import functools

import jax
import jax.numpy as jnp
from jax import lax
from jax.experimental import pallas as pl
from jax.experimental.pallas import tpu as pltpu
from jax.experimental.pallas import tpu_sc as plsc

_ROWS = 2
_NBUF = 4


@functools.cache
def _make_lookup(batch, seq, V, D):
    info = plsc.get_sparse_core_info()
    nc, ns = info.num_cores, info.num_subcores
    nw = nc * ns
    rows_per_w = batch // nw
    n_super = rows_per_w // (_ROWS * _NBUF)
    n_half = n_super // 2
    assert rows_per_w == n_half * 2 * _ROWS * _NBUF
    mesh = plsc.VectorSubcoreMesh(core_axis_name="c", subcore_axis_name="s")

    @functools.partial(
        pl.kernel,
        out_type=jax.ShapeDtypeStruct((batch, seq, 128), jnp.float32),
        mesh=mesh,
        scratch_types=[
            pltpu.VMEM((2, _NBUF, _ROWS, seq), jnp.int32),
            pltpu.VMEM((_NBUF, _ROWS, seq, D), jnp.float32),
            pltpu.SemaphoreType.DMA((2, _NBUF)),
            pltpu.SemaphoreType.DMA((_NBUF,)),
            pltpu.SemaphoreType.DMA((_NBUF,)),
        ],
        compiler_params=pltpu.CompilerParams(use_tc_tiling_on_sc=False),
    )
    def lookup(ids_hbm, table_hbm, out_hbm, idx_v, rows_v, isem, gsem, wsem):
        wid = lax.axis_index("s") * nc + lax.axis_index("c")
        base = wid * rows_per_w

        def chunk_row(i, b):
            return base + (i * _NBUF + b) * _ROWS

        def prefetch_ids(i, p):
            for b in range(_NBUF):
                pltpu.async_copy(
                    ids_hbm.at[pl.ds(chunk_row(i, b), _ROWS)],
                    idx_v.at[p, b],
                    isem.at[p, b],
                )

        prefetch_ids(0, 0)

        def super_pair(j, carry):
            for p in range(2):
                i = 2 * j + p
                for b in range(_NBUF):
                    row = chunk_row(i, b)

                    @pl.when(i > 0)
                    def _drain(b=b, row=row):
                        pltpu.make_async_copy(
                            rows_v.at[b],
                            out_hbm.at[pl.ds(row, _ROWS), :, pl.ds(0, D)],
                            wsem.at[b],
                        ).wait()

                    pltpu.make_async_copy(
                        ids_hbm.at[pl.ds(row, _ROWS)], idx_v.at[p, b], isem.at[p, b]
                    ).wait()
                    for r in range(_ROWS):
                        pltpu.async_copy(
                            table_hbm.at[idx_v.at[p, b, r]], rows_v.at[b, r], gsem.at[b]
                        )
                if p == 0:
                    prefetch_ids(i + 1, 1)
                else:

                    @pl.when(j < n_half - 1)
                    def _next_ids(i=i):
                        prefetch_ids(i + 1, 0)

                for b in range(_NBUF):
                    row = chunk_row(i, b)
                    pltpu.make_async_copy(
                        table_hbm.at[idx_v.at[p, b, 0]], rows_v.at[b], gsem.at[b]
                    ).wait()
                    pltpu.async_copy(
                        rows_v.at[b],
                        out_hbm.at[pl.ds(row, _ROWS), :, pl.ds(0, D)],
                        wsem.at[b],
                    )
            return carry

        lax.fori_loop(0, n_half, super_pair, 0)
        for b in range(_NBUF):
            pltpu.make_async_copy(
                rows_v.at[b],
                out_hbm.at[pl.ds(base, _ROWS), :, pl.ds(0, D)],
                wsem.at[b],
            ).wait()

    return lookup


def kernel(token_ids, weights):
    batch, seq = token_ids.shape
    vocab, d = weights.shape
    ids = token_ids.astype(jnp.int32)
    out_pad = _make_lookup(batch, seq, vocab, d)(ids, weights)
    return out_pad[:, :, :d]
